# baseline (speedup 1.0000x reference)
import functools

import jax
import jax.numpy as jnp
from jax import lax
from jax.experimental import pallas as pl
from jax.experimental.pallas import tpu as pltpu
from jax.experimental.pallas import tpu_sc as plsc

_ROWS = 8192
_DIM = 1024
_NW = 32
_RPW = _ROWS // _NW
_SC_CHUNK = 16
_NCH = _RPW // _SC_CHUNK
_NBUF = 8


@functools.partial(
    pl.kernel,
    mesh=plsc.VectorSubcoreMesh(core_axis_name="c", subcore_axis_name="s"),
    out_type=jax.ShapeDtypeStruct((_ROWS, _DIM), jnp.float32),
    scratch_types=[
        pltpu.VMEM_SHARED((16, _NBUF, _SC_CHUNK, _DIM), jnp.float32),
        pltpu.SemaphoreType.DMA((_NBUF,)),
        pltpu.SemaphoreType.DMA((_NBUF,)),
    ],
    compiler_params=pltpu.CompilerParams(
        skip_device_barrier=True,
        disable_bounds_checks=True,
        disable_semaphore_checks=True,
    ),
)
def _sc_copy(emb_hbm, out_hbm, sbuf, in_sems, out_sems):
    sid = lax.axis_index("s")
    wid = sid * 2 + lax.axis_index("c")
    base = wid * _RPW
    buf = sbuf.at[sid]

    def in_copy(g):
        slot = g % _NBUF
        return pltpu.make_async_copy(
            emb_hbm.at[pl.ds(base + g * _SC_CHUNK, _SC_CHUNK), :],
            buf.at[slot],
            in_sems.at[slot],
        )

    def out_copy(g):
        slot = g % _NBUF
        return pltpu.make_async_copy(
            buf.at[slot],
            out_hbm.at[pl.ds(base + g * _SC_CHUNK, _SC_CHUNK), :],
            out_sems.at[slot],
        )

    in_copy(0).start()
    for g in range(_NCH):
        in_copy(g).wait()
        out_copy(g).start()
        if g + 1 < _NCH:
            if g + 1 >= _NBUF:
                out_copy(g + 1 - _NBUF).wait()
            in_copy(g + 1).start()
    for g in range(max(0, _NCH - _NBUF), _NCH):
        out_copy(g).wait()


def kernel(x, emb):
    del x
    return _sc_copy(emb)


# speedup vs baseline: 1.1048x; 1.1048x over previous
import functools

import jax
import jax.numpy as jnp
from jax import lax
from jax.experimental import pallas as pl
from jax.experimental.pallas import tpu as pltpu
from jax.experimental.pallas import tpu_sc as plsc

_ROWS = 8192
_DIM = 1024
_NW = 32
_RPW = _ROWS // _NW
_SC_CHUNK = 32
_NCH = _RPW // _SC_CHUNK
_NBUF = 4


@functools.partial(
    pl.kernel,
    mesh=plsc.VectorSubcoreMesh(core_axis_name="c", subcore_axis_name="s"),
    out_type=jax.ShapeDtypeStruct((_ROWS, _DIM), jnp.float32),
    scratch_types=[
        pltpu.VMEM_SHARED((16, _NBUF, _SC_CHUNK, _DIM), jnp.float32),
        pltpu.SemaphoreType.DMA((_NBUF,)),
        pltpu.SemaphoreType.DMA((_NBUF,)),
    ],
    compiler_params=pltpu.CompilerParams(
        skip_device_barrier=True,
        disable_bounds_checks=True,
        disable_semaphore_checks=True,
    ),
)
def _sc_copy(emb_hbm, out_hbm, sbuf, in_sems, out_sems):
    sid = lax.axis_index("s")
    wid = sid * 2 + lax.axis_index("c")
    base = wid * _RPW
    buf = sbuf.at[sid]

    def in_copy(g):
        slot = g % _NBUF
        return pltpu.make_async_copy(
            emb_hbm.at[pl.ds(base + g * _SC_CHUNK, _SC_CHUNK), :],
            buf.at[slot],
            in_sems.at[slot],
        )

    def out_copy(g):
        slot = g % _NBUF
        return pltpu.make_async_copy(
            buf.at[slot],
            out_hbm.at[pl.ds(base + g * _SC_CHUNK, _SC_CHUNK), :],
            out_sems.at[slot],
        )

    in_copy(0).start()
    for g in range(_NCH):
        in_copy(g).wait()
        out_copy(g).start()
        if g + 1 < _NCH:
            if g + 1 >= _NBUF:
                out_copy(g + 1 - _NBUF).wait()
            in_copy(g + 1).start()
    for g in range(max(0, _NCH - _NBUF), _NCH):
        out_copy(g).wait()


def kernel(x, emb):
    del x
    return _sc_copy(emb)
